# baseline (speedup 1.0000x reference)
import functools

import jax
import jax.numpy as jnp
from jax import lax
from jax.experimental import pallas as pl
from jax.experimental.pallas import tpu as pltpu
from jax.experimental.pallas import tpu_sc as plsc

N_NODES = 10000
NP = 10240
E_EDGES = 160000
IN_DIM = 256
OUT_DIM = 512
NH = 8
HD = 64
NC, NS, L = 2, 16, 16
NW = NC * NS
EB1 = 64
EB2 = 64
NBLK1 = E_EDGES // EB1
NBLK2 = E_EDGES // EB2
ROWS_PER_TILE = NP // NS

_mesh = plsc.VectorSubcoreMesh(
    core_axis_name="c", subcore_axis_name="s", num_cores=NC, num_subcores=NS
)


def _matmul_body(h_ref, w_ref, o_ref):
    o_ref[...] = jnp.dot(h_ref[...], w_ref[...],
                         preferred_element_type=jnp.float32)


def _fused_matmul(hP, w_cat):
    RB = 512
    return pl.pallas_call(
        _matmul_body,
        grid=(NP // RB,),
        in_specs=[
            pl.BlockSpec((RB, IN_DIM), lambda i: (i, 0)),
            pl.BlockSpec((IN_DIM, 1152), lambda i: (0, 0)),
        ],
        out_specs=pl.BlockSpec((RB, 1152), lambda i: (i, 0)),
        out_shape=jax.ShapeDtypeStruct((NP, 1152), jnp.float32),
    )(hP, w_cat)


@functools.partial(
    pl.kernel,
    out_type=jax.ShapeDtypeStruct((E_EDGES // 8, 128), jnp.float32),
    mesh=_mesh,
    scratch_types=[
        pltpu.VMEM((EB1,), jnp.int32),
        pltpu.VMEM((EB1,), jnp.int32),
        pltpu.VMEM((EB1, 128), jnp.float32),
        pltpu.VMEM((EB1, 128), jnp.float32),
        pltpu.VMEM((EB1 // 8, 128), jnp.float32),
        pltpu.SemaphoreType.DMA,
        pltpu.SemaphoreType.DMA,
    ],
)
def _pass1(s_hbm, src_hbm, dst_hbm, alpha_hbm,
           src_v, dst_v, ssrc_v, sdst_v, a_v, sem1, sem2):
    c = lax.axis_index("c")
    s = lax.axis_index("s")
    wid = s * NC + c

    base_cnt = NBLK1 // NW
    rem = NBLK1 - base_cnt * NW
    cnt = base_cnt + jnp.where(wid < rem, 1, 0)
    start = wid * base_cnt + jnp.minimum(wid, rem)

    def block_body(j, carry):
        blk = start + j
        e0 = blk * EB1
        pltpu.sync_copy(src_hbm.at[pl.ds(e0, EB1)], src_v)
        pltpu.sync_copy(dst_hbm.at[pl.ds(e0, EB1)], dst_v)
        cp1 = pltpu.async_copy(s_hbm.at[src_v], ssrc_v, sem1)
        cp2 = pltpu.async_copy(s_hbm.at[dst_v], sdst_v, sem2)
        cp1.wait()
        cp2.wait()

        def edge_body(e, carry2):
            x = ssrc_v[e, pl.ds(0, L)]
            y = sdst_v[e, pl.ds(L, L)]
            ev = x + y
            al = jnp.exp(jnp.where(ev >= 0, ev, 0.2 * ev))
            a_v[e // 8, pl.ds((e % 8) * L, L)] = al
            return carry2

        lax.fori_loop(0, EB1, edge_body, 0)
        pltpu.sync_copy(a_v, alpha_hbm.at[pl.ds(blk * (EB1 // 8), EB1 // 8)])
        return carry

    lax.fori_loop(0, cnt, block_body, 0)


@functools.partial(
    pl.kernel,
    out_type=jax.ShapeDtypeStruct((NC, 5, NP, 128), jnp.float32),
    mesh=_mesh,
    scratch_types=[
        pltpu.VMEM((EB2,), jnp.int32),
        pltpu.VMEM((EB2,), jnp.int32),
        pltpu.VMEM((EB2 // 8, 128), jnp.float32),
        pltpu.VMEM((EB2, 128), jnp.float32),
        pltpu.VMEM((EB2,), jnp.int32),
        pltpu.VMEM((EB2,), jnp.int32),
        pltpu.VMEM((EB2 // 8, 128), jnp.float32),
        pltpu.VMEM((EB2, 128), jnp.float32),
        pltpu.VMEM_SHARED((NP, 128), jnp.float32),
        pltpu.SemaphoreType.DMA,
        pltpu.SemaphoreType.DMA,
    ],
)
def _pass2(hp0, hp1, hp2, hp3, snd_hbm, rcv_hbm, alpha_hbm, out_hbm,
           sidx0, ridx0, al0, rows0, sidx1, ridx1, al1, rows1, acc_sp,
           sem0, sem1):
    c = lax.axis_index("c")
    s = lax.axis_index("s")

    base_cnt = NBLK2 // NS
    rem = NBLK2 - base_cnt * NS
    cnt = base_cnt + jnp.where(s < rem, 1, 0)
    start = s * base_cnt + jnp.minimum(s, rem)
    r0 = s * ROWS_PER_TILE
    zv = jnp.zeros((L,), jnp.float32)
    slots = ((sidx0, ridx0, al0, rows0, sem0),
             (sidx1, ridx1, al1, rows1, sem1))

    def load_idx(blk, slot):
        sidx, ridx, al, _, _ = slot
        e0 = blk * EB2
        pltpu.sync_copy(snd_hbm.at[c, pl.ds(e0, EB2)], sidx)
        pltpu.sync_copy(rcv_hbm.at[c, pl.ds(e0, EB2)], ridx)
        pltpu.sync_copy(alpha_hbm.at[pl.ds(blk * (EB2 // 8), EB2 // 8)], al)

    for k, hp in enumerate((hp0, hp1, hp2, hp3, None)):

        def zrow_body(e, carry):
            for g in range(128 // L):
                rows0[e, pl.ds(g * L, L)] = zv
            return carry

        lax.fori_loop(0, EB2, zrow_body, 0)
        for z in range(ROWS_PER_TILE // EB2):
            pltpu.sync_copy(rows0, acc_sp.at[pl.ds(r0 + z * EB2, EB2)])
        plsc.subcore_barrier()

        if hp is not None:
            load_idx(start, slots[0])
            pltpu.async_copy(hp.at[slots[0][0]], slots[0][3], sem0)

            def pair_body(jj, carry):
                for b in (0, 1):
                    j = jj * 2 + b
                    cur = slots[b]
                    nxt = slots[1 - b]

                    @pl.when(j < cnt)
                    def _():
                        @pl.when(j + 1 < cnt)
                        def _():
                            load_idx(start + j + 1, nxt)
                            pltpu.async_copy(hp.at[nxt[0]], nxt[3], nxt[4])

                        pltpu.make_async_copy(
                            hp.at[cur[0]], cur[3], cur[4]).wait()
                        al_v = cur[2]
                        rows_v = cur[3]

                        def edge_body(e, carry2):
                            av = al_v[e // 8, pl.ds((e % 8) * L, L)]
                            a0 = jnp.where(c == 0, av[2 * k],
                                           av[NH + 2 * k])
                            a1 = jnp.where(c == 0, av[2 * k + 1],
                                           av[NH + 2 * k + 1])
                            for g in range(8):
                                sc = a0 if g < 4 else a1
                                rows_v[e, pl.ds(g * L, L)] = (
                                    rows_v[e, pl.ds(g * L, L)] * sc)
                            return carry2

                        lax.fori_loop(0, EB2, edge_body, 0)
                        pltpu.sync_copy(rows_v, acc_sp.at[cur[1]], add=True)

                return carry

            lax.fori_loop(0, (cnt + 1) // 2, pair_body, 0)

        else:
            def block_body(j, carry):
                blk = start + j
                e0 = blk * EB2
                pltpu.sync_copy(rcv_hbm.at[c, pl.ds(e0, EB2)], ridx0)
                pltpu.sync_copy(
                    alpha_hbm.at[pl.ds(blk * (EB2 // 8), EB2 // 8)], al0)

                def edge_body(e, carry2):
                    rows0[e, pl.ds(0, L)] = al0[e // 8,
                                                pl.ds((e % 8) * L, L)]
                    return carry2

                lax.fori_loop(0, EB2, edge_body, 0)
                pltpu.sync_copy(rows0, acc_sp.at[ridx0], add=True)
                return carry

            lax.fori_loop(0, cnt, block_body, 0)

        plsc.subcore_barrier()
        for z in range(ROWS_PER_TILE // EB2):
            pltpu.sync_copy(acc_sp.at[pl.ds(r0 + z * EB2, EB2)], rows0)
            pltpu.sync_copy(rows0, out_hbm.at[c, k, pl.ds(r0 + z * EB2, EB2)])
        plsc.subcore_barrier()


def _final_body(hs_ref, ai_ref, ao_ref, a4_ref, b_ref, g_ref, be_ref, o_ref):
    ain = a4_ref[0, :, :NH]
    aout = a4_ref[1, :, NH:2 * NH]
    rin = 1.0 / (ain + 1e-8)
    rout = 1.0 / (aout + 1e-8)
    col = lax.broadcasted_iota(jnp.int32, (NH, OUT_DIM), 1) // HD
    row = lax.broadcasted_iota(jnp.int32, (NH, OUT_DIM), 0)
    expand = (col == row).astype(jnp.float32)
    rin_e = jnp.dot(rin, expand, preferred_element_type=jnp.float32)
    rout_e = jnp.dot(rout, expand, preferred_element_type=jnp.float32)
    comb = (hs_ref[...] + ai_ref[...] * rin_e + ao_ref[...] * rout_e
            + b_ref[...])
    mu = jnp.mean(comb, axis=-1, keepdims=True)
    var = jnp.mean((comb - mu) ** 2, axis=-1, keepdims=True)
    o_ref[...] = (comb - mu) * lax.rsqrt(var + 1e-5) * g_ref[...] + be_ref[...]


def _finalize(hsP, acc_in, acc_out, a4, bias2, gamma2, beta2):
    RB = 512
    return pl.pallas_call(
        _final_body,
        grid=(NP // RB,),
        in_specs=[
            pl.BlockSpec((RB, OUT_DIM), lambda i: (i, 0)),
            pl.BlockSpec((RB, OUT_DIM), lambda i: (i, 0)),
            pl.BlockSpec((RB, OUT_DIM), lambda i: (i, 0)),
            pl.BlockSpec((NC, RB, 128), lambda i: (0, i, 0)),
            pl.BlockSpec((1, OUT_DIM), lambda i: (0, 0)),
            pl.BlockSpec((1, OUT_DIM), lambda i: (0, 0)),
            pl.BlockSpec((1, OUT_DIM), lambda i: (0, 0)),
        ],
        out_specs=pl.BlockSpec((RB, OUT_DIM), lambda i: (i, 0)),
        out_shape=jax.ShapeDtypeStruct((NP, OUT_DIM), jnp.float32),
    )(hsP, acc_in, acc_out, a4, bias2, gamma2, beta2)


def kernel(h, edge_index, W, W_self, a_in, a_out, bias, gamma, beta):
    n = h.shape[0]
    eye = jnp.eye(NH, dtype=jnp.float32)

    def block_diag(m):
        return (eye[:, None, :] * m[:, :, None]).reshape(OUT_DIM, NH)

    A = jnp.concatenate([
        block_diag(a_in[:, :HD]), block_diag(a_out[:, HD:]),
        block_diag(a_in[:, HD:]), block_diag(a_out[:, :HD]),
    ], axis=1)
    w_cat = jnp.concatenate([W, W_self, W @ A], axis=1)

    hP = jnp.zeros((NP, IN_DIM), jnp.float32).at[:n].set(h)
    mm = _fused_matmul(hP, w_cat)
    hps = tuple(mm[:, k * 128:(k + 1) * 128] for k in range(4))
    hsP = mm[:, OUT_DIM:2 * OUT_DIM]
    s_scal = jnp.zeros((NP, 128), jnp.float32).at[:, :32].set(
        mm[:, 2 * OUT_DIM:2 * OUT_DIM + 32])

    src = edge_index[0]
    dst = edge_index[1]
    snd = jnp.stack([src, dst])
    rcv = jnp.stack([dst, src])

    alpha = _pass1(s_scal, src, dst)
    acc = _pass2(*hps, snd, rcv, alpha)

    acc_in = acc[0, :4].transpose(1, 0, 2).reshape(NP, OUT_DIM)
    acc_out = acc[1, :4].transpose(1, 0, 2).reshape(NP, OUT_DIM)
    out = _finalize(hsP, acc_in, acc_out, acc[:, 4],
                    bias[None, :], gamma[None, :], beta[None, :])
    return out[:n]


# speedup vs baseline: 1.0028x; 1.0028x over previous
import functools

import jax
import jax.numpy as jnp
from jax import lax
from jax.experimental import pallas as pl
from jax.experimental.pallas import tpu as pltpu
from jax.experimental.pallas import tpu_sc as plsc

N_NODES = 10000
NP = 10240
E_EDGES = 160000
IN_DIM = 256
OUT_DIM = 512
NH = 8
HD = 64
NC, NS, L = 2, 16, 16
NW = NC * NS
EB1 = 64
EB2 = 64
NBLK1 = E_EDGES // EB1
NBLK2 = E_EDGES // EB2
ROWS_PER_TILE = NP // NS

_mesh = plsc.VectorSubcoreMesh(
    core_axis_name="c", subcore_axis_name="s", num_cores=NC, num_subcores=NS
)


def _matmul_body(h_ref, w_ref, o_ref):
    o_ref[...] = jnp.dot(h_ref[...], w_ref[...],
                         preferred_element_type=jnp.float32)


def _fused_matmul(hP, w_cat):
    RB = 512
    return pl.pallas_call(
        _matmul_body,
        grid=(NP // RB,),
        in_specs=[
            pl.BlockSpec((RB, IN_DIM), lambda i: (i, 0)),
            pl.BlockSpec((IN_DIM, 1152), lambda i: (0, 0)),
        ],
        out_specs=pl.BlockSpec((RB, 1152), lambda i: (i, 0)),
        out_shape=jax.ShapeDtypeStruct((NP, 1152), jnp.float32),
    )(hP, w_cat)


@functools.partial(
    pl.kernel,
    out_type=jax.ShapeDtypeStruct((E_EDGES // 8, 128), jnp.float32),
    mesh=_mesh,
    scratch_types=[
        pltpu.VMEM((EB1,), jnp.int32),
        pltpu.VMEM((EB1,), jnp.int32),
        pltpu.VMEM((EB1, 128), jnp.float32),
        pltpu.VMEM((EB1, 128), jnp.float32),
        pltpu.VMEM((EB1 // 8, 128), jnp.float32),
        pltpu.SemaphoreType.DMA,
        pltpu.SemaphoreType.DMA,
    ],
)
def _pass1(s_hbm, src_hbm, dst_hbm, alpha_hbm,
           src_v, dst_v, ssrc_v, sdst_v, a_v, sem1, sem2):
    c = lax.axis_index("c")
    s = lax.axis_index("s")
    wid = s * NC + c

    base_cnt = NBLK1 // NW
    rem = NBLK1 - base_cnt * NW
    cnt = base_cnt + jnp.where(wid < rem, 1, 0)
    start = wid * base_cnt + jnp.minimum(wid, rem)

    def block_body(j, carry):
        blk = start + j
        e0 = blk * EB1
        pltpu.sync_copy(src_hbm.at[pl.ds(e0, EB1)], src_v)
        pltpu.sync_copy(dst_hbm.at[pl.ds(e0, EB1)], dst_v)
        cp1 = pltpu.async_copy(s_hbm.at[src_v], ssrc_v, sem1)
        cp2 = pltpu.async_copy(s_hbm.at[dst_v], sdst_v, sem2)
        cp1.wait()
        cp2.wait()

        def edge_body(e, carry2):
            x = ssrc_v[e, pl.ds(0, L)]
            y = sdst_v[e, pl.ds(L, L)]
            ev = x + y
            al = jnp.exp(jnp.where(ev >= 0, ev, 0.2 * ev))
            a_v[e // 8, pl.ds((e % 8) * L, L)] = al
            return carry2

        lax.fori_loop(0, EB1, edge_body, 0)
        pltpu.sync_copy(a_v, alpha_hbm.at[pl.ds(blk * (EB1 // 8), EB1 // 8)])
        return carry

    lax.fori_loop(0, cnt, block_body, 0)


@functools.partial(
    pl.kernel,
    out_type=jax.ShapeDtypeStruct((NC, 5, NP, 128), jnp.float32),
    mesh=_mesh,
    scratch_types=[
        pltpu.VMEM((EB2,), jnp.int32),
        pltpu.VMEM((EB2,), jnp.int32),
        pltpu.VMEM((EB2 // 8, 128), jnp.float32),
        pltpu.VMEM((EB2, 128), jnp.float32),
        pltpu.VMEM((EB2,), jnp.int32),
        pltpu.VMEM((EB2,), jnp.int32),
        pltpu.VMEM((EB2 // 8, 128), jnp.float32),
        pltpu.VMEM((EB2, 128), jnp.float32),
        pltpu.VMEM_SHARED((NP, 128), jnp.float32),
        pltpu.SemaphoreType.DMA,
        pltpu.SemaphoreType.DMA,
        pltpu.SemaphoreType.DMA,
        pltpu.SemaphoreType.DMA,
    ],
)
def _pass2(hp0, hp1, hp2, hp3, snd_hbm, rcv_hbm, alpha_hbm, out_hbm,
           sidx0, ridx0, al0, rows0, sidx1, ridx1, al1, rows1, acc_sp,
           sem0, sem1, ssem0, ssem1):
    c = lax.axis_index("c")
    s = lax.axis_index("s")

    base_cnt = NBLK2 // NS
    rem = NBLK2 - base_cnt * NS
    cnt = base_cnt + jnp.where(s < rem, 1, 0)
    start = s * base_cnt + jnp.minimum(s, rem)
    r0 = s * ROWS_PER_TILE
    zv = jnp.zeros((L,), jnp.float32)
    slots = ((sidx0, ridx0, al0, rows0, sem0, ssem0),
             (sidx1, ridx1, al1, rows1, sem1, ssem1))

    def load_idx(blk, slot):
        sidx, ridx, al = slot[0], slot[1], slot[2]
        e0 = blk * EB2
        pltpu.sync_copy(snd_hbm.at[c, pl.ds(e0, EB2)], sidx)
        pltpu.sync_copy(rcv_hbm.at[c, pl.ds(e0, EB2)], ridx)
        pltpu.sync_copy(alpha_hbm.at[pl.ds(blk * (EB2 // 8), EB2 // 8)], al)

    for k, hp in enumerate((hp0, hp1, hp2, hp3, None)):

        def zrow_body(e, carry):
            for g in range(128 // L):
                rows0[e, pl.ds(g * L, L)] = zv
            return carry

        lax.fori_loop(0, EB2, zrow_body, 0)
        for z in range(ROWS_PER_TILE // EB2):
            pltpu.sync_copy(rows0, acc_sp.at[pl.ds(r0 + z * EB2, EB2)])
        plsc.subcore_barrier()

        if hp is not None:
            load_idx(start, slots[0])
            pltpu.async_copy(hp.at[slots[0][0]], slots[0][3], sem0)

            def pair_body(jj, carry):
                for b in (0, 1):
                    j = jj * 2 + b
                    cur = slots[b]
                    nxt = slots[1 - b]

                    @pl.when(j < cnt)
                    def _():
                        @pl.when(j + 1 < cnt)
                        def _():
                            @pl.when(j >= 1)
                            def _():
                                pltpu.make_async_copy(
                                    nxt[3], acc_sp.at[pl.ds(0, EB2)],
                                    nxt[5]).wait()

                            load_idx(start + j + 1, nxt)
                            pltpu.async_copy(hp.at[nxt[0]], nxt[3], nxt[4])

                        pltpu.make_async_copy(
                            hp.at[cur[0]], cur[3], cur[4]).wait()
                        al_v = cur[2]
                        rows_v = cur[3]

                        def edge_body(e, carry2):
                            av = al_v[e // 8, pl.ds((e % 8) * L, L)]
                            a0 = jnp.where(c == 0, av[2 * k],
                                           av[NH + 2 * k])
                            a1 = jnp.where(c == 0, av[2 * k + 1],
                                           av[NH + 2 * k + 1])
                            for g in range(8):
                                sc = a0 if g < 4 else a1
                                rows_v[e, pl.ds(g * L, L)] = (
                                    rows_v[e, pl.ds(g * L, L)] * sc)
                            return carry2

                        lax.fori_loop(0, EB2, edge_body, 0)
                        pltpu.async_copy(rows_v, acc_sp.at[cur[1]],
                                         cur[5], add=True)

                return carry

            lax.fori_loop(0, (cnt + 1) // 2, pair_body, 0)
            for b in (0, 1):
                pltpu.make_async_copy(
                    slots[b][3], acc_sp.at[pl.ds(0, EB2)],
                    slots[b][5]).wait()

        else:
            def block_body(j, carry):
                blk = start + j
                e0 = blk * EB2
                pltpu.sync_copy(rcv_hbm.at[c, pl.ds(e0, EB2)], ridx0)
                pltpu.sync_copy(
                    alpha_hbm.at[pl.ds(blk * (EB2 // 8), EB2 // 8)], al0)

                def edge_body(e, carry2):
                    rows0[e, pl.ds(0, L)] = al0[e // 8,
                                                pl.ds((e % 8) * L, L)]
                    return carry2

                lax.fori_loop(0, EB2, edge_body, 0)
                pltpu.sync_copy(rows0, acc_sp.at[ridx0], add=True)
                return carry

            lax.fori_loop(0, cnt, block_body, 0)

        plsc.subcore_barrier()
        for z in range(ROWS_PER_TILE // EB2):
            pltpu.sync_copy(acc_sp.at[pl.ds(r0 + z * EB2, EB2)], rows0)
            pltpu.sync_copy(rows0, out_hbm.at[c, k, pl.ds(r0 + z * EB2, EB2)])
        plsc.subcore_barrier()


def _final_body(hs_ref, ai_ref, ao_ref, a4_ref, b_ref, g_ref, be_ref, o_ref):
    ain = a4_ref[0, :, :NH]
    aout = a4_ref[1, :, NH:2 * NH]
    rin = 1.0 / (ain + 1e-8)
    rout = 1.0 / (aout + 1e-8)
    col = lax.broadcasted_iota(jnp.int32, (NH, OUT_DIM), 1) // HD
    row = lax.broadcasted_iota(jnp.int32, (NH, OUT_DIM), 0)
    expand = (col == row).astype(jnp.float32)
    rin_e = jnp.dot(rin, expand, preferred_element_type=jnp.float32)
    rout_e = jnp.dot(rout, expand, preferred_element_type=jnp.float32)
    comb = (hs_ref[...] + ai_ref[...] * rin_e + ao_ref[...] * rout_e
            + b_ref[...])
    mu = jnp.mean(comb, axis=-1, keepdims=True)
    var = jnp.mean((comb - mu) ** 2, axis=-1, keepdims=True)
    o_ref[...] = (comb - mu) * lax.rsqrt(var + 1e-5) * g_ref[...] + be_ref[...]


def _finalize(hsP, acc_in, acc_out, a4, bias2, gamma2, beta2):
    RB = 512
    return pl.pallas_call(
        _final_body,
        grid=(NP // RB,),
        in_specs=[
            pl.BlockSpec((RB, OUT_DIM), lambda i: (i, 0)),
            pl.BlockSpec((RB, OUT_DIM), lambda i: (i, 0)),
            pl.BlockSpec((RB, OUT_DIM), lambda i: (i, 0)),
            pl.BlockSpec((NC, RB, 128), lambda i: (0, i, 0)),
            pl.BlockSpec((1, OUT_DIM), lambda i: (0, 0)),
            pl.BlockSpec((1, OUT_DIM), lambda i: (0, 0)),
            pl.BlockSpec((1, OUT_DIM), lambda i: (0, 0)),
        ],
        out_specs=pl.BlockSpec((RB, OUT_DIM), lambda i: (i, 0)),
        out_shape=jax.ShapeDtypeStruct((NP, OUT_DIM), jnp.float32),
    )(hsP, acc_in, acc_out, a4, bias2, gamma2, beta2)


def kernel(h, edge_index, W, W_self, a_in, a_out, bias, gamma, beta):
    n = h.shape[0]
    eye = jnp.eye(NH, dtype=jnp.float32)

    def block_diag(m):
        return (eye[:, None, :] * m[:, :, None]).reshape(OUT_DIM, NH)

    A = jnp.concatenate([
        block_diag(a_in[:, :HD]), block_diag(a_out[:, HD:]),
        block_diag(a_in[:, HD:]), block_diag(a_out[:, :HD]),
    ], axis=1)
    w_cat = jnp.concatenate([W, W_self, W @ A], axis=1)

    hP = jnp.zeros((NP, IN_DIM), jnp.float32).at[:n].set(h)
    mm = _fused_matmul(hP, w_cat)
    hps = tuple(mm[:, k * 128:(k + 1) * 128] for k in range(4))
    hsP = mm[:, OUT_DIM:2 * OUT_DIM]
    s_scal = jnp.zeros((NP, 128), jnp.float32).at[:, :32].set(
        mm[:, 2 * OUT_DIM:2 * OUT_DIM + 32])

    src = edge_index[0]
    dst = edge_index[1]
    snd = jnp.stack([src, dst])
    rcv = jnp.stack([dst, src])

    alpha = _pass1(s_scal, src, dst)
    acc = _pass2(*hps, snd, rcv, alpha)

    acc_in = acc[0, :4].transpose(1, 0, 2).reshape(NP, OUT_DIM)
    acc_out = acc[1, :4].transpose(1, 0, 2).reshape(NP, OUT_DIM)
    out = _finalize(hsP, acc_in, acc_out, acc[:, 4],
                    bias[None, :], gamma[None, :], beta[None, :])
    return out[:n]


# speedup vs baseline: 1.1433x; 1.1400x over previous
import functools

import jax
import jax.numpy as jnp
from jax import lax
from jax.experimental import pallas as pl
from jax.experimental.pallas import tpu as pltpu
from jax.experimental.pallas import tpu_sc as plsc

N_NODES = 10000
NP = 10240
E_EDGES = 160000
IN_DIM = 256
OUT_DIM = 512
NH = 8
HD = 64
NC, NS, L = 2, 16, 16
NW = NC * NS
EB1 = 64
EB2 = 64
NBLK1 = E_EDGES // EB1
NBLK2 = E_EDGES // EB2
ROWS_PER_TILE = NP // NS

_mesh = plsc.VectorSubcoreMesh(
    core_axis_name="c", subcore_axis_name="s", num_cores=NC, num_subcores=NS
)


def _matmul_body(h_ref, w_ref, o_ref):
    o_ref[...] = jnp.dot(h_ref[...], w_ref[...],
                         preferred_element_type=jnp.float32)


def _fused_matmul(hP, w_cat):
    RB = 512
    return pl.pallas_call(
        _matmul_body,
        grid=(NP // RB,),
        in_specs=[
            pl.BlockSpec((RB, IN_DIM), lambda i: (i, 0)),
            pl.BlockSpec((IN_DIM, 1152), lambda i: (0, 0)),
        ],
        out_specs=pl.BlockSpec((RB, 1152), lambda i: (i, 0)),
        out_shape=jax.ShapeDtypeStruct((NP, 1152), jnp.float32),
    )(hP, w_cat)


@functools.partial(
    pl.kernel,
    out_type=jax.ShapeDtypeStruct((2, E_EDGES // 8, 128), jnp.float32),
    mesh=_mesh,
    scratch_types=[
        pltpu.VMEM((EB1,), jnp.int32),
        pltpu.VMEM((EB1,), jnp.int32),
        pltpu.VMEM((EB1, 128), jnp.float32),
        pltpu.VMEM((EB1, 128), jnp.float32),
        pltpu.VMEM((EB1 // 8, 128), jnp.float32),
        pltpu.VMEM((EB1 // 8, 128), jnp.float32),
        pltpu.SemaphoreType.DMA,
        pltpu.SemaphoreType.DMA,
    ],
)
def _pass1(s_hbm, src_hbm, dst_hbm, alpha_hbm,
           src_v, dst_v, ssrc_v, sdst_v, a_v, a2_v, sem1, sem2):
    c = lax.axis_index("c")
    s = lax.axis_index("s")
    wid = s * NC + c

    base_cnt = NBLK1 // NW
    rem = NBLK1 - base_cnt * NW
    cnt = base_cnt + jnp.where(wid < rem, 1, 0)
    start = wid * base_cnt + jnp.minimum(wid, rem)

    def block_body(j, carry):
        blk = start + j
        e0 = blk * EB1
        pltpu.sync_copy(src_hbm.at[pl.ds(e0, EB1)], src_v)
        pltpu.sync_copy(dst_hbm.at[pl.ds(e0, EB1)], dst_v)
        cp1 = pltpu.async_copy(s_hbm.at[src_v], ssrc_v, sem1)
        cp2 = pltpu.async_copy(s_hbm.at[dst_v], sdst_v, sem2)
        cp1.wait()
        cp2.wait()

        def row_body(e8, carry2):
            for eo in range(8):
                e = e8 * 8 + eo
                ev = ssrc_v[e, pl.ds(0, L)] + sdst_v[e, pl.ds(L, L)]
                a_v[e8, pl.ds(eo * L, L)] = jnp.exp(
                    jnp.maximum(ev, 0.2 * ev))
                ev2 = ssrc_v[e, pl.ds(2 * L, L)] + sdst_v[e, pl.ds(3 * L, L)]
                a2_v[e8, pl.ds(eo * L, L)] = jnp.exp(
                    jnp.maximum(ev2, 0.2 * ev2))
            return carry2

        lax.fori_loop(0, EB1 // 8, row_body, 0)
        pltpu.sync_copy(a_v,
                        alpha_hbm.at[0, pl.ds(blk * (EB1 // 8), EB1 // 8)])
        pltpu.sync_copy(a2_v,
                        alpha_hbm.at[1, pl.ds(blk * (EB1 // 8), EB1 // 8)])
        return carry

    lax.fori_loop(0, cnt, block_body, 0)


@functools.partial(
    pl.kernel,
    out_type=jax.ShapeDtypeStruct((NC, 5, NP, 128), jnp.float32),
    mesh=_mesh,
    scratch_types=[
        pltpu.VMEM((EB2,), jnp.int32),
        pltpu.VMEM((EB2,), jnp.int32),
        pltpu.VMEM((EB2 // 8, 128), jnp.float32),
        pltpu.VMEM((EB2, 128), jnp.float32),
        pltpu.VMEM((EB2,), jnp.int32),
        pltpu.VMEM((EB2,), jnp.int32),
        pltpu.VMEM((EB2 // 8, 128), jnp.float32),
        pltpu.VMEM((EB2, 128), jnp.float32),
        pltpu.VMEM_SHARED((NP, 128), jnp.float32),
        pltpu.SemaphoreType.DMA,
        pltpu.SemaphoreType.DMA,
        pltpu.SemaphoreType.DMA,
        pltpu.SemaphoreType.DMA,
    ],
)
def _pass2(hp0, hp1, hp2, hp3, snd_hbm, rcv_hbm, alpha_hbm, out_hbm,
           sidx0, ridx0, al0, rows0, sidx1, ridx1, al1, rows1, acc_sp,
           sem0, sem1, ssem0, ssem1):
    c = lax.axis_index("c")
    s = lax.axis_index("s")

    base_cnt = NBLK2 // NS
    rem = NBLK2 - base_cnt * NS
    cnt = base_cnt + jnp.where(s < rem, 1, 0)
    start = s * base_cnt + jnp.minimum(s, rem)
    r0 = s * ROWS_PER_TILE
    zv = jnp.zeros((L,), jnp.float32)
    slots = ((sidx0, ridx0, al0, rows0, sem0, ssem0),
             (sidx1, ridx1, al1, rows1, sem1, ssem1))

    def load_idx(blk, slot):
        sidx, ridx, al = slot[0], slot[1], slot[2]
        e0 = blk * EB2
        pltpu.sync_copy(snd_hbm.at[c, pl.ds(e0, EB2)], sidx)
        pltpu.sync_copy(rcv_hbm.at[c, pl.ds(e0, EB2)], ridx)
        pltpu.sync_copy(alpha_hbm.at[c, pl.ds(blk * (EB2 // 8), EB2 // 8)],
                        al)

    for k, hp in enumerate((hp0, hp1, hp2, hp3, None)):

        def zrow_body(e, carry):
            for g in range(128 // L):
                rows0[e, pl.ds(g * L, L)] = zv
            return carry

        lax.fori_loop(0, EB2, zrow_body, 0)
        for z in range(ROWS_PER_TILE // EB2):
            pltpu.sync_copy(rows0, acc_sp.at[pl.ds(r0 + z * EB2, EB2)])
        plsc.subcore_barrier()

        if hp is not None:
            load_idx(start, slots[0])
            pltpu.async_copy(hp.at[slots[0][0]], slots[0][3], sem0)

            def pair_body(jj, carry):
                for b in (0, 1):
                    j = jj * 2 + b
                    cur = slots[b]
                    nxt = slots[1 - b]

                    @pl.when(j < cnt)
                    def _():
                        @pl.when(j + 1 < cnt)
                        def _():
                            @pl.when(j >= 1)
                            def _():
                                pltpu.make_async_copy(
                                    nxt[3], acc_sp.at[pl.ds(0, EB2)],
                                    nxt[5]).wait()

                            load_idx(start + j + 1, nxt)
                            pltpu.async_copy(hp.at[nxt[0]], nxt[3], nxt[4])

                        pltpu.make_async_copy(
                            hp.at[cur[0]], cur[3], cur[4]).wait()
                        al_v = cur[2]
                        rows_v = cur[3]

                        def row_body(e8, carry2):
                            for eo in range(8):
                                av = al_v[e8, pl.ds(eo * L, L)]
                                a0 = av[2 * k]
                                a1 = av[2 * k + 1]
                                e = e8 * 8 + eo
                                for g in range(8):
                                    sc = a0 if g < 4 else a1
                                    rows_v[e, pl.ds(g * L, L)] = (
                                        rows_v[e, pl.ds(g * L, L)] * sc)
                            return carry2

                        lax.fori_loop(0, EB2 // 8, row_body, 0)
                        pltpu.async_copy(rows_v, acc_sp.at[cur[1]],
                                         cur[5], add=True)

                return carry

            lax.fori_loop(0, (cnt + 1) // 2, pair_body, 0)
            for b in (0, 1):
                pltpu.make_async_copy(
                    slots[b][3], acc_sp.at[pl.ds(0, EB2)],
                    slots[b][5]).wait()

        else:
            def block_body(j, carry):
                blk = start + j
                e0 = blk * EB2
                pltpu.sync_copy(rcv_hbm.at[c, pl.ds(e0, EB2)], ridx0)
                pltpu.sync_copy(
                    alpha_hbm.at[c, pl.ds(blk * (EB2 // 8), EB2 // 8)], al0)

                def row_body(e8, carry2):
                    for eo in range(8):
                        rows0[e8 * 8 + eo, pl.ds(0, L)] = al0[
                            e8, pl.ds(eo * L, L)]
                    return carry2

                lax.fori_loop(0, EB2 // 8, row_body, 0)
                pltpu.sync_copy(rows0, acc_sp.at[ridx0], add=True)
                return carry

            lax.fori_loop(0, cnt, block_body, 0)

        plsc.subcore_barrier()
        for z in range(ROWS_PER_TILE // EB2):
            pltpu.sync_copy(acc_sp.at[pl.ds(r0 + z * EB2, EB2)], rows0)
            pltpu.sync_copy(rows0, out_hbm.at[c, k, pl.ds(r0 + z * EB2, EB2)])
        plsc.subcore_barrier()


def _final_body(hs_ref, ai_ref, ao_ref, a4_ref, b_ref, g_ref, be_ref, o_ref):
    ain = a4_ref[0, :, :NH]
    aout = a4_ref[1, :, :NH]
    rin = 1.0 / (ain + 1e-8)
    rout = 1.0 / (aout + 1e-8)
    col = lax.broadcasted_iota(jnp.int32, (NH, OUT_DIM), 1) // HD
    row = lax.broadcasted_iota(jnp.int32, (NH, OUT_DIM), 0)
    expand = (col == row).astype(jnp.float32)
    rin_e = jnp.dot(rin, expand, preferred_element_type=jnp.float32)
    rout_e = jnp.dot(rout, expand, preferred_element_type=jnp.float32)
    comb = (hs_ref[...] + ai_ref[...] * rin_e + ao_ref[...] * rout_e
            + b_ref[...])
    mu = jnp.mean(comb, axis=-1, keepdims=True)
    var = jnp.mean((comb - mu) ** 2, axis=-1, keepdims=True)
    o_ref[...] = (comb - mu) * lax.rsqrt(var + 1e-5) * g_ref[...] + be_ref[...]


def _finalize(hsP, acc_in, acc_out, a4, bias2, gamma2, beta2):
    RB = 512
    return pl.pallas_call(
        _final_body,
        grid=(NP // RB,),
        in_specs=[
            pl.BlockSpec((RB, OUT_DIM), lambda i: (i, 0)),
            pl.BlockSpec((RB, OUT_DIM), lambda i: (i, 0)),
            pl.BlockSpec((RB, OUT_DIM), lambda i: (i, 0)),
            pl.BlockSpec((NC, RB, 128), lambda i: (0, i, 0)),
            pl.BlockSpec((1, OUT_DIM), lambda i: (0, 0)),
            pl.BlockSpec((1, OUT_DIM), lambda i: (0, 0)),
            pl.BlockSpec((1, OUT_DIM), lambda i: (0, 0)),
        ],
        out_specs=pl.BlockSpec((RB, OUT_DIM), lambda i: (i, 0)),
        out_shape=jax.ShapeDtypeStruct((NP, OUT_DIM), jnp.float32),
    )(hsP, acc_in, acc_out, a4, bias2, gamma2, beta2)


def kernel(h, edge_index, W, W_self, a_in, a_out, bias, gamma, beta):
    n = h.shape[0]
    eye = jnp.eye(NH, dtype=jnp.float32)

    def block_diag(m):
        return (eye[:, None, :] * m[:, :, None]).reshape(OUT_DIM, NH)

    A = jnp.concatenate([
        block_diag(a_in[:, :HD]), block_diag(a_out[:, HD:]),
        block_diag(a_in[:, HD:]), block_diag(a_out[:, :HD]),
        block_diag(a_out[:, HD:]), block_diag(a_in[:, :HD]),
        block_diag(a_out[:, :HD]), block_diag(a_in[:, HD:]),
    ], axis=1)
    w_cat = jnp.concatenate([W, W_self, W @ A], axis=1)

    hP = jnp.zeros((NP, IN_DIM), jnp.float32).at[:n].set(h)
    mm = _fused_matmul(hP, w_cat)
    hps = tuple(mm[:, k * 128:(k + 1) * 128] for k in range(4))
    hsP = mm[:, OUT_DIM:2 * OUT_DIM]
    s_scal = jnp.zeros((NP, 128), jnp.float32).at[:, :64].set(
        mm[:, 2 * OUT_DIM:2 * OUT_DIM + 64])

    src = edge_index[0]
    dst = edge_index[1]
    snd = jnp.stack([src, dst])
    rcv = jnp.stack([dst, src])

    alpha = _pass1(s_scal, src, dst)
    acc = _pass2(*hps, snd, rcv, alpha)

    acc_in = acc[0, :4].transpose(1, 0, 2).reshape(NP, OUT_DIM)
    acc_out = acc[1, :4].transpose(1, 0, 2).reshape(NP, OUT_DIM)
    out = _finalize(hsP, acc_in, acc_out, acc[:, 4],
                    bias[None, :], gamma[None, :], beta[None, :])
    return out[:n]


# speedup vs baseline: 1.6636x; 1.4552x over previous
import functools

import jax
import jax.numpy as jnp
from jax import lax
from jax.experimental import pallas as pl
from jax.experimental.pallas import tpu as pltpu
from jax.experimental.pallas import tpu_sc as plsc

N_NODES = 10000
NP = 10240
E_EDGES = 160000
IN_DIM = 256
OUT_DIM = 512
NH = 8
HD = 64
NC, NS, L = 2, 16, 16
NW = NC * NS
EB1 = 64
EB2 = 64
NBLK1 = E_EDGES // EB1
NBLK2 = E_EDGES // EB2
ROWS_PER_TILE = NP // NS

_mesh = plsc.VectorSubcoreMesh(
    core_axis_name="c", subcore_axis_name="s", num_cores=NC, num_subcores=NS
)


def _matmul_body(h_ref, w_ref, o_ref):
    o_ref[...] = jnp.dot(h_ref[...], w_ref[...],
                         preferred_element_type=jnp.float32)


def _fused_matmul(hP, w_cat):
    RB = 512
    return pl.pallas_call(
        _matmul_body,
        grid=(NP // RB,),
        in_specs=[
            pl.BlockSpec((RB, IN_DIM), lambda i: (i, 0)),
            pl.BlockSpec((IN_DIM, 1152), lambda i: (0, 0)),
        ],
        out_specs=pl.BlockSpec((RB, 1152), lambda i: (i, 0)),
        out_shape=jax.ShapeDtypeStruct((NP, 1152), jnp.float32),
    )(hP, w_cat)


@functools.partial(
    pl.kernel,
    out_type=jax.ShapeDtypeStruct((2, E_EDGES // 8, 128), jnp.float32),
    mesh=_mesh,
    scratch_types=[
        pltpu.VMEM((EB1,), jnp.int32),
        pltpu.VMEM((EB1,), jnp.int32),
        pltpu.VMEM((EB1, 128), jnp.float32),
        pltpu.VMEM((EB1, 128), jnp.float32),
        pltpu.VMEM((EB1 // 8, 128), jnp.float32),
        pltpu.VMEM((EB1 // 8, 128), jnp.float32),
        pltpu.SemaphoreType.DMA,
        pltpu.SemaphoreType.DMA,
    ],
)
def _pass1(s_hbm, src_hbm, dst_hbm, alpha_hbm,
           src_v, dst_v, ssrc_v, sdst_v, a_v, a2_v, sem1, sem2):
    c = lax.axis_index("c")
    s = lax.axis_index("s")
    wid = s * NC + c

    base_cnt = NBLK1 // NW
    rem = NBLK1 - base_cnt * NW
    cnt = base_cnt + jnp.where(wid < rem, 1, 0)
    start = wid * base_cnt + jnp.minimum(wid, rem)

    def block_body(j, carry):
        blk = start + j
        e0 = blk * EB1
        cpa = pltpu.async_copy(src_hbm.at[pl.ds(e0, EB1)], src_v, sem1)
        cpb = pltpu.async_copy(dst_hbm.at[pl.ds(e0, EB1)], dst_v, sem2)
        cpa.wait()
        cpb.wait()
        cp1 = pltpu.async_copy(s_hbm.at[src_v], ssrc_v, sem1)
        cp2 = pltpu.async_copy(s_hbm.at[dst_v], sdst_v, sem2)
        cp1.wait()
        cp2.wait()

        def row_body(e8, carry2):
            for eo in range(8):
                e = e8 * 8 + eo
                ev = ssrc_v[e, pl.ds(0, L)] + sdst_v[e, pl.ds(L, L)]
                a_v[e8, pl.ds(eo * L, L)] = jnp.exp(
                    jnp.maximum(ev, 0.2 * ev))
                ev2 = ssrc_v[e, pl.ds(2 * L, L)] + sdst_v[e, pl.ds(3 * L, L)]
                a2_v[e8, pl.ds(eo * L, L)] = jnp.exp(
                    jnp.maximum(ev2, 0.2 * ev2))
            return carry2

        lax.fori_loop(0, EB1 // 8, row_body, 0)
        cw1 = pltpu.async_copy(
            a_v, alpha_hbm.at[0, pl.ds(blk * (EB1 // 8), EB1 // 8)], sem1)
        cw2 = pltpu.async_copy(
            a2_v, alpha_hbm.at[1, pl.ds(blk * (EB1 // 8), EB1 // 8)], sem2)
        cw1.wait()
        cw2.wait()
        return carry

    lax.fori_loop(0, cnt, block_body, 0)


@functools.partial(
    pl.kernel,
    out_type=jax.ShapeDtypeStruct((NC, 5, NP, 128), jnp.float32),
    mesh=_mesh,
    scratch_types=[
        pltpu.VMEM((EB2,), jnp.int32),
        pltpu.VMEM((EB2,), jnp.int32),
        pltpu.VMEM((EB2 // 8, 128), jnp.float32),
        pltpu.VMEM((EB2, 128), jnp.float32),
        pltpu.VMEM((EB2,), jnp.int32),
        pltpu.VMEM((EB2,), jnp.int32),
        pltpu.VMEM((EB2 // 8, 128), jnp.float32),
        pltpu.VMEM((EB2, 128), jnp.float32),
        pltpu.VMEM_SHARED((NP, 128), jnp.float32),
        pltpu.SemaphoreType.DMA,
        pltpu.SemaphoreType.DMA,
        pltpu.SemaphoreType.DMA,
        pltpu.SemaphoreType.DMA,
        pltpu.SemaphoreType.DMA,
        pltpu.SemaphoreType.DMA,
        pltpu.SemaphoreType.DMA,
        pltpu.SemaphoreType.DMA,
        pltpu.SemaphoreType.DMA,
        pltpu.SemaphoreType.DMA,
    ],
)
def _pass2(hp0, hp1, hp2, hp3, snd_hbm, rcv_hbm, alpha_hbm, out_hbm,
           sidx0, ridx0, al0, rows0, sidx1, ridx1, al1, rows1, acc_sp,
           sem0, sem1, ssem0, ssem1,
           isem0s, isem0r, isem0a, isem1s, isem1r, isem1a):
    c = lax.axis_index("c")
    s = lax.axis_index("s")

    base_cnt = NBLK2 // NS
    rem = NBLK2 - base_cnt * NS
    cnt = base_cnt + jnp.where(s < rem, 1, 0)
    start = s * base_cnt + jnp.minimum(s, rem)
    r0 = s * ROWS_PER_TILE
    zv = jnp.zeros((L,), jnp.float32)
    slots = ((sidx0, ridx0, al0, rows0, sem0, ssem0, isem0s, isem0r, isem0a),
             (sidx1, ridx1, al1, rows1, sem1, ssem1, isem1s, isem1r, isem1a))

    def issue_idx(blk, slot):
        e0 = blk * EB2
        pltpu.async_copy(snd_hbm.at[c, pl.ds(e0, EB2)], slot[0], slot[6])
        pltpu.async_copy(rcv_hbm.at[c, pl.ds(e0, EB2)], slot[1], slot[7])
        pltpu.async_copy(alpha_hbm.at[c, pl.ds(blk * (EB2 // 8), EB2 // 8)],
                         slot[2], slot[8])

    def wait_snd(slot):
        pltpu.make_async_copy(
            snd_hbm.at[c, pl.ds(0, EB2)], slot[0], slot[6]).wait()

    def wait_rcv_al(slot):
        pltpu.make_async_copy(
            rcv_hbm.at[c, pl.ds(0, EB2)], slot[1], slot[7]).wait()
        pltpu.make_async_copy(
            alpha_hbm.at[c, pl.ds(0, EB2 // 8)], slot[2], slot[8]).wait()

    for k, hp in enumerate((hp0, hp1, hp2, hp3, None)):

        def zrow_body(e, carry):
            for g in range(128 // L):
                rows0[e, pl.ds(g * L, L)] = zv
            return carry

        lax.fori_loop(0, EB2, zrow_body, 0)
        for z in range(ROWS_PER_TILE // EB2):
            pltpu.sync_copy(rows0, acc_sp.at[pl.ds(r0 + z * EB2, EB2)])
        plsc.subcore_barrier()

        if hp is not None:
            issue_idx(start, slots[0])
            wait_snd(slots[0])
            pltpu.async_copy(hp.at[slots[0][0]], slots[0][3], sem0)

            def pair_body(jj, carry):
                for b in (0, 1):
                    j = jj * 2 + b
                    cur = slots[b]
                    nxt = slots[1 - b]

                    @pl.when(j < cnt)
                    def _():
                        @pl.when(j + 1 < cnt)
                        def _():
                            @pl.when(j >= 1)
                            def _():
                                pltpu.make_async_copy(
                                    nxt[3], acc_sp.at[pl.ds(0, EB2)],
                                    nxt[5]).wait()

                            issue_idx(start + j + 1, nxt)
                            wait_snd(nxt)
                            pltpu.async_copy(hp.at[nxt[0]], nxt[3], nxt[4])

                        pltpu.make_async_copy(
                            hp.at[cur[0]], cur[3], cur[4]).wait()
                        wait_rcv_al(cur)
                        al_v = cur[2]
                        rows_v = cur[3]

                        def row_body(e8, carry2):
                            for eo in range(8):
                                av = al_v[e8, pl.ds(eo * L, L)]
                                a0 = av[2 * k]
                                a1 = av[2 * k + 1]
                                e = e8 * 8 + eo
                                for g in range(8):
                                    sc = a0 if g < 4 else a1
                                    rows_v[e, pl.ds(g * L, L)] = (
                                        rows_v[e, pl.ds(g * L, L)] * sc)
                            return carry2

                        lax.fori_loop(0, EB2 // 8, row_body, 0)
                        pltpu.async_copy(rows_v, acc_sp.at[cur[1]],
                                         cur[5], add=True)

                return carry

            lax.fori_loop(0, (cnt + 1) // 2, pair_body, 0)
            for b in (0, 1):
                pltpu.make_async_copy(
                    slots[b][3], acc_sp.at[pl.ds(0, EB2)],
                    slots[b][5]).wait()

        else:
            def block_body(j, carry):
                blk = start + j
                e0 = blk * EB2
                cpr = pltpu.async_copy(
                    rcv_hbm.at[c, pl.ds(e0, EB2)], ridx0, isem0r)
                cpl = pltpu.async_copy(
                    alpha_hbm.at[c, pl.ds(blk * (EB2 // 8), EB2 // 8)], al0,
                    isem0a)
                cpr.wait()
                cpl.wait()

                def row_body(e8, carry2):
                    for eo in range(8):
                        rows0[e8 * 8 + eo, pl.ds(0, L)] = al0[
                            e8, pl.ds(eo * L, L)]
                    return carry2

                lax.fori_loop(0, EB2 // 8, row_body, 0)
                pltpu.sync_copy(rows0, acc_sp.at[ridx0], add=True)
                return carry

            lax.fori_loop(0, cnt, block_body, 0)

        plsc.subcore_barrier()
        for z in range(ROWS_PER_TILE // EB2):
            pltpu.sync_copy(acc_sp.at[pl.ds(r0 + z * EB2, EB2)], rows0)
            pltpu.sync_copy(rows0, out_hbm.at[c, k, pl.ds(r0 + z * EB2, EB2)])
        plsc.subcore_barrier()


def _final_body(hs_ref, ai_ref, ao_ref, a4_ref, b_ref, g_ref, be_ref, o_ref):
    ain = a4_ref[0, :, :NH]
    aout = a4_ref[1, :, :NH]
    rin = 1.0 / (ain + 1e-8)
    rout = 1.0 / (aout + 1e-8)
    col = lax.broadcasted_iota(jnp.int32, (NH, OUT_DIM), 1) // HD
    row = lax.broadcasted_iota(jnp.int32, (NH, OUT_DIM), 0)
    expand = (col == row).astype(jnp.float32)
    rin_e = jnp.dot(rin, expand, preferred_element_type=jnp.float32)
    rout_e = jnp.dot(rout, expand, preferred_element_type=jnp.float32)
    comb = (hs_ref[...] + ai_ref[...] * rin_e + ao_ref[...] * rout_e
            + b_ref[...])
    mu = jnp.mean(comb, axis=-1, keepdims=True)
    var = jnp.mean((comb - mu) ** 2, axis=-1, keepdims=True)
    o_ref[...] = (comb - mu) * lax.rsqrt(var + 1e-5) * g_ref[...] + be_ref[...]


def _finalize(hsP, acc_in, acc_out, a4, bias2, gamma2, beta2):
    RB = 512
    return pl.pallas_call(
        _final_body,
        grid=(NP // RB,),
        in_specs=[
            pl.BlockSpec((RB, OUT_DIM), lambda i: (i, 0)),
            pl.BlockSpec((RB, OUT_DIM), lambda i: (i, 0)),
            pl.BlockSpec((RB, OUT_DIM), lambda i: (i, 0)),
            pl.BlockSpec((NC, RB, 128), lambda i: (0, i, 0)),
            pl.BlockSpec((1, OUT_DIM), lambda i: (0, 0)),
            pl.BlockSpec((1, OUT_DIM), lambda i: (0, 0)),
            pl.BlockSpec((1, OUT_DIM), lambda i: (0, 0)),
        ],
        out_specs=pl.BlockSpec((RB, OUT_DIM), lambda i: (i, 0)),
        out_shape=jax.ShapeDtypeStruct((NP, OUT_DIM), jnp.float32),
    )(hsP, acc_in, acc_out, a4, bias2, gamma2, beta2)


def kernel(h, edge_index, W, W_self, a_in, a_out, bias, gamma, beta):
    n = h.shape[0]
    eye = jnp.eye(NH, dtype=jnp.float32)

    def block_diag(m):
        return (eye[:, None, :] * m[:, :, None]).reshape(OUT_DIM, NH)

    A = jnp.concatenate([
        block_diag(a_in[:, :HD]), block_diag(a_out[:, HD:]),
        block_diag(a_in[:, HD:]), block_diag(a_out[:, :HD]),
        block_diag(a_out[:, HD:]), block_diag(a_in[:, :HD]),
        block_diag(a_out[:, :HD]), block_diag(a_in[:, HD:]),
    ], axis=1)
    w_cat = jnp.concatenate([W, W_self, W @ A], axis=1)

    hP = jnp.zeros((NP, IN_DIM), jnp.float32).at[:n].set(h)
    mm = _fused_matmul(hP, w_cat)
    hps = tuple(mm[:, k * 128:(k + 1) * 128] for k in range(4))
    hsP = mm[:, OUT_DIM:2 * OUT_DIM]
    s_scal = jnp.zeros((NP, 128), jnp.float32).at[:, :64].set(
        mm[:, 2 * OUT_DIM:2 * OUT_DIM + 64])

    src = edge_index[0]
    dst = edge_index[1]
    snd = jnp.stack([src, dst])
    rcv = jnp.stack([dst, src])

    alpha = _pass1(s_scal, src, dst)
    acc = _pass2(*hps, snd, rcv, alpha)

    acc_in = acc[0, :4].transpose(1, 0, 2).reshape(NP, OUT_DIM)
    acc_out = acc[1, :4].transpose(1, 0, 2).reshape(NP, OUT_DIM)
    out = _finalize(hsP, acc_in, acc_out, acc[:, 4],
                    bias[None, :], gamma[None, :], beta[None, :])
    return out[:n]
